# Initial kernel scaffold; baseline (speedup 1.0000x reference)
import jax
import jax.numpy as jnp
from jax.experimental import pallas as pl


def kernel(x, params):
    raise NotImplementedError("write your pallas kernel here")


# speedup vs baseline: 9.5434x; 9.5434x over previous
import functools
import numpy as np
import jax
import jax.numpy as jnp
from jax.experimental import pallas as pl
from jax.experimental.pallas import tpu as pltpu

B, IN_CH, H, W = 1, 202, 256, 256
NF = 128
N_ITERS = 10
N_SPIX = 2621

_nw = int(np.sqrt(N_SPIX * W / H) + 0.5)
_nh = int(np.sqrt(N_SPIX * H / W) + 0.5)
NW, NH = _nw, _nh
K = NW * NH
WS, HS = W / NW, H / NH

_cx_of_x = np.clip((np.arange(W) / WS).astype(np.int32), 0, NW - 1)
_cy_of_y = np.clip((np.arange(H) / HS).astype(np.int32), 0, NH - 1)
_wx = np.bincount(_cx_of_x, minlength=NW)
_wy = np.bincount(_cy_of_y, minlength=NH)
assert _wx[0] == 6 and np.all(_wx[1:] == 5), _wx
assert _wy[0] == 6 and np.all(_wy[1:] == 5), _wy
SLOT = 6
S = SLOT * SLOT

_mrow = np.zeros((NH, SLOT), np.float32)
for _i in range(NH):
    _mrow[_i, : _wy[_i]] = 1.0
_mcol = np.zeros((NW, SLOT), np.float32)
for _i in range(NW):
    _mcol[_i, : _wx[_i]] = 1.0
MASK_SK = np.einsum('yr,xs->rsyx', _mrow, _mcol).reshape(S, K)

_cyk, _cxk = np.arange(K) // NW, np.arange(K) % NW
VALID_9K = np.zeros((9, K), np.float32)
OFFS = []
_j = 0
for _dy in (-1, 0, 1):
    for _dx in (-1, 0, 1):
        VALID_9K[_j] = (((_cyk + _dy) >= 0) & ((_cyk + _dy) < NH)
                        & ((_cxk + _dx) >= 0) & ((_cxk + _dx) < NW))
        OFFS.append(_dy * NW + _dx)
        _j += 1

_yv, _xv = np.meshgrid(np.arange(H, dtype=np.float32),
                       np.arange(W, dtype=np.float32), indexing='ij')
COORDS = np.stack([_xv * NW / W, _yv * NH / H])[None]

C_TOT = NF + 2

_LEAK = 0.01


def _leaky(v):
    return jnp.where(v >= 0, v, _LEAK * v)


PIX = H * W
PBLK = 4096
NPB = PIX // PBLK


def _a1_body(x_ref, w0_ref, b0_ref, w1_ref, b1_ref, w2_ref, b2_ref,
             w3_ref, b3_ref, s1_ref, t2_ref, t3_ref):
    xb = x_ref[...]
    s = _leaky(jnp.dot(w0_ref[...], xb, preferred_element_type=jnp.float32)
               + b0_ref[...])
    s1_ref[...] = _leaky(
        jnp.dot(w1_ref[...], s, preferred_element_type=jnp.float32)
        + b1_ref[...])
    t2_ref[...] = (jnp.dot(w2_ref[...], s, preferred_element_type=jnp.float32)
                   + b2_ref[...])
    t3_ref[...] = (jnp.dot(w3_ref[...], s, preferred_element_type=jnp.float32)
                   + b3_ref[...])


def _run_a1(x2d, w0, b0, w1, b1, w2, b2, w3, b3):
    full = lambda shape: pl.BlockSpec(shape, lambda i: (0, 0))
    return pl.pallas_call(
        _a1_body,
        grid=(NPB,),
        in_specs=[
            pl.BlockSpec((IN_CH, PBLK), lambda i: (0, i)),
            full((NF, IN_CH)), full((NF, 1)),
            full((NF, NF)), full((NF, 1)),
            full((NF, NF)), full((NF, 1)),
            full((NF, NF)), full((NF, 1)),
        ],
        out_specs=[pl.BlockSpec((NF, PBLK), lambda i: (0, i))] * 3,
        out_shape=[jax.ShapeDtypeStruct((NF, PIX), jnp.float32)] * 3,
        compiler_params=pltpu.CompilerParams(
            vmem_limit_bytes=100 * 1024 * 1024),
    )(x2d, w0, b0, w1, b1, w2, b2, w3, b3)


CBLK = 16
NCB = NF // CBLK


def _a2_body(s1_ref, t2_ref, t3_ref, w3x3_ref, b3x3_ref, w5x5_ref, b5x5_ref,
             sig_ref, out_ref, scr3, scr5):
    scr3[...] = jnp.zeros_like(scr3)
    scr5[...] = jnp.zeros_like(scr5)
    scr3[:, 1:H + 1, 1:W + 1] = t2_ref[...]
    scr5[:, 2:H + 2, 2:W + 2] = t3_ref[...]
    cbase = pl.program_id(0) * CBLK
    sig0, sig1, sig2 = sig_ref[0], sig_ref[1], sig_ref[2]

    def cbody(c, carry):
        cg = cbase + c
        acc3 = jnp.zeros((H, W), jnp.float32)
        for u in range(3):
            for v in range(3):
                acc3 = acc3 + scr3[c, u:u + H, v:v + W] * w3x3_ref[cg, u * 3 + v]
        s2c = _leaky(acc3 + b3x3_ref[cg])
        acc5 = jnp.zeros((H, W), jnp.float32)
        for u in range(5):
            for v in range(5):
                acc5 = acc5 + scr5[c, u:u + H, v:v + W] * w5x5_ref[cg, u * 5 + v]
        s3c = _leaky(acc5 + b5x5_ref[cg])
        out_ref[c] = sig0 * s1_ref[c] + sig1 * s2c + sig2 * s3c
        return carry

    jax.lax.fori_loop(0, CBLK, cbody, 0)


def _run_a2(s1, t2, t3, w3x3, b3x3, w5x5, b5x5, sig):
    img = lambda: pl.BlockSpec((CBLK, H, W), lambda i: (i, 0, 0))
    smem = lambda shape: pl.BlockSpec(
        shape, lambda i: tuple(0 for _ in shape), memory_space=pltpu.SMEM)
    return pl.pallas_call(
        _a2_body,
        grid=(NCB,),
        in_specs=[
            img(), img(), img(),
            smem((NF, 9)), smem((NF,)), smem((NF, 25)), smem((NF,)),
            smem((3,)),
        ],
        out_specs=img(),
        out_shape=jax.ShapeDtypeStruct((NF, H, W), jnp.float32),
        scratch_shapes=[
            pltpu.VMEM((CBLK, H + 2, W + 2), jnp.float32),
            pltpu.VMEM((CBLK, H + 4, W + 4), jnp.float32),
        ],
        compiler_params=pltpu.CompilerParams(
            vmem_limit_bytes=100 * 1024 * 1024),
    )(s1, t2, t3, w3x3, b3x3, w5x5, b5x5, sig)


def _shift(v, off):
    if off == 0:
        return v
    if off > 0:
        return jnp.concatenate(
            [v[..., off:], jnp.zeros_like(v[..., :off])], axis=-1)
    return jnp.concatenate(
        [jnp.zeros_like(v[..., off:]), v[..., :off]], axis=-1)


def _b_body(pfp_ref, mask_ref, valid_ref, q_ref, spf_ref):
    mask = mask_ref[...]
    cnt = jnp.sum(mask, axis=0, keepdims=True)
    inv0 = 1.0 / jnp.maximum(cnt, 1.0)

    def init_c(c, carry):
        row = jnp.sum(pfp_ref[c], axis=0, keepdims=True) * inv0
        spf_ref[pl.ds(c, 1), :] = row
        return carry

    jax.lax.fori_loop(0, C_TOT, init_c, 0)

    def p2_c(c, acc):
        pc = pfp_ref[c]
        return acc + pc * pc

    p2 = jax.lax.fori_loop(0, C_TOT, p2_c, jnp.zeros((S, K), jnp.float32))

    def one_iter(it, carry):
        def s2_c(c, acc):
            row = spf_ref[pl.ds(c, 1), :]
            return acc + row * row

        s2 = jax.lax.fori_loop(0, C_TOT, s2_c, jnp.zeros((1, K), jnp.float32))

        m = None
        for j in range(9):
            def dot_c(c, acc, _j=j):
                return acc + pfp_ref[c] * _shift(
                    spf_ref[pl.ds(c, 1), :], OFFS[_j])

            dot = jax.lax.fori_loop(0, C_TOT, dot_c,
                                    jnp.zeros((S, K), jnp.float32))
            lj = jnp.where(valid_ref[j:j + 1, :] > 0,
                           2.0 * dot - p2 - _shift(s2, OFFS[j]), -1e16)
            q_ref[j] = lj
            m = lj if m is None else jnp.maximum(m, lj)

        z = jnp.zeros((S, K), jnp.float32)
        for j in range(9):
            e = jnp.exp(q_ref[j] - m)
            q_ref[j] = e
            z = z + e
        invz = 1.0 / z
        for j in range(9):
            q_ref[j] = q_ref[j] * invz * mask

        den = jnp.zeros((1, K), jnp.float32)
        for j in range(9):
            den = den + _shift(jnp.sum(q_ref[j], axis=0, keepdims=True),
                               -OFFS[j])
        invden = 1.0 / (den + 1e-16)

        def num_c(c, carry2):
            pc = pfp_ref[c]
            row = jnp.zeros((1, K), jnp.float32)
            for j in range(9):
                row = row + _shift(
                    jnp.sum(pc * q_ref[j], axis=0, keepdims=True), -OFFS[j])
            spf_ref[pl.ds(c, 1), :] = row * invden
            return carry2

        jax.lax.fori_loop(0, C_TOT, num_c, 0)
        return carry

    jax.lax.fori_loop(0, N_ITERS, one_iter, 0)


def _run_b(pfp, mask, valid):
    return pl.pallas_call(
        _b_body,
        out_shape=[
            jax.ShapeDtypeStruct((9, S, K), jnp.float32),
            jax.ShapeDtypeStruct((C_TOT, K), jnp.float32),
        ],
        compiler_params=pltpu.CompilerParams(
            vmem_limit_bytes=120 * 1024 * 1024),
    )(pfp, mask, valid)


def _pad_axis_to_slots(a, axis):
    n = a.shape[axis]
    first = jax.lax.slice_in_dim(a, 0, SLOT, axis=axis)
    rest = jax.lax.slice_in_dim(a, SLOT, n, axis=axis)
    shp = list(rest.shape)
    shp[axis:axis + 1] = [NW - 1, 5]
    rest = rest.reshape(shp)
    pad = [(0, 0, 0)] * rest.ndim
    pad[axis + 1] = (0, 1, 0)
    rest = jax.lax.pad(rest, jnp.float32(0), pad)
    shp2 = list(rest.shape)
    shp2[axis:axis + 2] = [(NW - 1) * SLOT]
    rest = rest.reshape(shp2)
    return jnp.concatenate([first, rest], axis=axis)


def _unpad_axis(a, axis):
    first = jax.lax.slice_in_dim(a, 0, SLOT, axis=axis)
    rest = jax.lax.slice_in_dim(a, SLOT, NW * SLOT, axis=axis)
    shp = list(rest.shape)
    shp[axis:axis + 1] = [NW - 1, SLOT]
    rest = rest.reshape(shp)
    rest = jax.lax.slice_in_dim(rest, 0, 5, axis=axis + 1)
    shp2 = list(rest.shape)
    shp2[axis:axis + 2] = [(NW - 1) * 5]
    rest = rest.reshape(shp2)
    return jnp.concatenate([first, rest], axis=axis)


def _to_slot_cell(img):
    p = _pad_axis_to_slots(img, 1)
    p = _pad_axis_to_slots(p, 2)
    c = img.shape[0]
    p = p.reshape(c, NH, SLOT, NW, SLOT)
    p = p.transpose(0, 2, 4, 1, 3)
    return p.reshape(c, S, K)


def _from_slot_cell(sc):
    j = sc.shape[0]
    p = sc.reshape(j, SLOT, SLOT, NH, NW)
    p = p.transpose(0, 3, 1, 4, 2)
    p = p.reshape(j, NH * SLOT, NW * SLOT)
    p = _unpad_axis(p, 1)
    return _unpad_axis(p, 2)


@jax.jit
def kernel(x, params):
    f32 = jnp.float32

    def fold(w, g, b):
        w2 = w[:, :, 0, 0]
        return (w2 * g[None, :]).astype(f32), (w2 @ b)[:, None].astype(f32)

    w0, b0 = fold(params['stem_w'], params['stem_bn_g'], params['stem_bn_b'])
    w1, b1 = fold(params['s1_w'], params['s1_bn_g'], params['s1_bn_b'])
    w2, b2 = fold(params['s2_pw_w'], params['s2_bn_g'], params['s2_bn_b'])
    w3, b3 = fold(params['s3_pw_w'], params['s3_bn_g'], params['s3_bn_b'])

    x2d = x.reshape(IN_CH, PIX).astype(f32)
    s1, t2, t3 = _run_a1(x2d, w0, b0, w1, b1, w2, b2, w3, b3)

    w3x3 = params['s2_dw_w'].reshape(NF, 9).astype(f32)
    w5x5 = params['s3_dw_w'].reshape(NF, 25).astype(f32)
    sig = jnp.concatenate([params['sigma0'], params['sigma1'],
                           params['sigma2']]).astype(f32)
    pf_img = _run_a2(s1.reshape(NF, H, W), t2.reshape(NF, H, W),
                     t3.reshape(NF, H, W), w3x3,
                     params['s2_dw_b'].astype(f32), w5x5,
                     params['s3_dw_b'].astype(f32), sig)

    pf130 = jnp.concatenate(
        [pf_img, jnp.asarray(COORDS[0], f32)], axis=0)
    pfp = _to_slot_cell(pf130)

    qp, spf = _run_b(pfp, jnp.asarray(MASK_SK, f32),
                     jnp.asarray(VALID_9K, f32))

    q_img = _from_slot_cell(qp)[None]
    spf_out = spf[None, :NF, :]
    return q_img, spf_out, pf_img[None]


# speedup vs baseline: 13.3130x; 1.3950x over previous
import functools
import numpy as np
import jax
import jax.numpy as jnp
from jax.experimental import pallas as pl
from jax.experimental.pallas import tpu as pltpu

B, IN_CH, H, W = 1, 202, 256, 256
NF = 128
N_ITERS = 10
N_SPIX = 2621

_nw = int(np.sqrt(N_SPIX * W / H) + 0.5)
_nh = int(np.sqrt(N_SPIX * H / W) + 0.5)
NW, NH = _nw, _nh
K = NW * NH
WS, HS = W / NW, H / NH

_cx_of_x = np.clip((np.arange(W) / WS).astype(np.int32), 0, NW - 1)
_cy_of_y = np.clip((np.arange(H) / HS).astype(np.int32), 0, NH - 1)
_wx = np.bincount(_cx_of_x, minlength=NW)
_wy = np.bincount(_cy_of_y, minlength=NH)
assert _wx[0] == 6 and np.all(_wx[1:] == 5), _wx
assert _wy[0] == 6 and np.all(_wy[1:] == 5), _wy
SLOT = 6
S = SLOT * SLOT

_mrow = np.zeros((NH, SLOT), np.float32)
for _i in range(NH):
    _mrow[_i, : _wy[_i]] = 1.0
_mcol = np.zeros((NW, SLOT), np.float32)
for _i in range(NW):
    _mcol[_i, : _wx[_i]] = 1.0
MASK_SK = np.einsum('yr,xs->rsyx', _mrow, _mcol).reshape(S, K)

_cyk, _cxk = np.arange(K) // NW, np.arange(K) % NW
VALID_9K = np.zeros((9, K), np.float32)
OFFS = []
_j = 0
for _dy in (-1, 0, 1):
    for _dx in (-1, 0, 1):
        VALID_9K[_j] = (((_cyk + _dy) >= 0) & ((_cyk + _dy) < NH)
                        & ((_cxk + _dx) >= 0) & ((_cxk + _dx) < NW))
        OFFS.append(_dy * NW + _dx)
        _j += 1

_ystart = np.zeros(NH, np.int64)
for _i in range(1, NH):
    _ystart[_i] = _ystart[_i - 1] + _wy[_i - 1]
_xstart = np.zeros(NW, np.int64)
for _i in range(1, NW):
    _xstart[_i] = _xstart[_i - 1] + _wx[_i - 1]
RY = np.zeros((SLOT, NH, H), np.float32)
for _cy in range(NH):
    for _r in range(_wy[_cy]):
        RY[_r, _cy, _ystart[_cy] + _r] = 1.0
RX = np.zeros((SLOT, NW, W), np.float32)
for _cx in range(NW):
    for _r in range(_wx[_cx]):
        RX[_r, _cx, _xstart[_cx] + _r] = 1.0

_yv, _xv = np.meshgrid(np.arange(H, dtype=np.float32),
                       np.arange(W, dtype=np.float32), indexing='ij')
COORDS = np.stack([_xv * NW / W, _yv * NH / H])[None]

C_TOT = NF + 2

_LEAK = 0.01


def _leaky(v):
    return jnp.where(v >= 0, v, _LEAK * v)


PIX = H * W
PBLK = 4096
NPB = PIX // PBLK


def _a1_body(x_ref, w0_ref, b0_ref, w1_ref, b1_ref, w2_ref, b2_ref,
             w3_ref, b3_ref, s1_ref, t2_ref, t3_ref):
    xb = x_ref[...]
    s = _leaky(jnp.dot(w0_ref[...], xb, preferred_element_type=jnp.float32,
                   precision=jax.lax.Precision.HIGHEST)
               + b0_ref[...])
    s1_ref[...] = _leaky(
        jnp.dot(w1_ref[...], s, preferred_element_type=jnp.float32,
                   precision=jax.lax.Precision.HIGHEST)
        + b1_ref[...])
    t2_ref[...] = (jnp.dot(w2_ref[...], s, preferred_element_type=jnp.float32,
                   precision=jax.lax.Precision.HIGHEST)
                   + b2_ref[...])
    t3_ref[...] = (jnp.dot(w3_ref[...], s, preferred_element_type=jnp.float32,
                   precision=jax.lax.Precision.HIGHEST)
                   + b3_ref[...])


def _run_a1(x2d, w0, b0, w1, b1, w2, b2, w3, b3):
    full = lambda shape: pl.BlockSpec(shape, lambda i: (0, 0))
    return pl.pallas_call(
        _a1_body,
        grid=(NPB,),
        in_specs=[
            pl.BlockSpec((IN_CH, PBLK), lambda i: (0, i)),
            full((NF, IN_CH)), full((NF, 1)),
            full((NF, NF)), full((NF, 1)),
            full((NF, NF)), full((NF, 1)),
            full((NF, NF)), full((NF, 1)),
        ],
        out_specs=[pl.BlockSpec((NF, PBLK), lambda i: (0, i))] * 3,
        out_shape=[jax.ShapeDtypeStruct((NF, PIX), jnp.float32)] * 3,
        compiler_params=pltpu.CompilerParams(
            vmem_limit_bytes=100 * 1024 * 1024),
    )(x2d, w0, b0, w1, b1, w2, b2, w3, b3)


CBLK = 16
NCB = NF // CBLK


def _a2_body(s1_ref, t2_ref, t3_ref, w3x3_ref, b3x3_ref, w5x5_ref, b5x5_ref,
             sig_ref, out_ref, scr3, scr5):
    scr3[...] = jnp.zeros_like(scr3)
    scr5[...] = jnp.zeros_like(scr5)
    scr3[:, 1:H + 1, 1:W + 1] = t2_ref[...]
    scr5[:, 2:H + 2, 2:W + 2] = t3_ref[...]
    cbase = pl.program_id(0) * CBLK
    sig0, sig1, sig2 = sig_ref[0], sig_ref[1], sig_ref[2]

    def cbody(c, carry):
        cg = cbase + c
        acc3 = jnp.zeros((H, W), jnp.float32)
        for u in range(3):
            for v in range(3):
                acc3 = acc3 + scr3[c, u:u + H, v:v + W] * w3x3_ref[cg, u * 3 + v]
        s2c = _leaky(acc3 + b3x3_ref[cg])
        acc5 = jnp.zeros((H, W), jnp.float32)
        for u in range(5):
            for v in range(5):
                acc5 = acc5 + scr5[c, u:u + H, v:v + W] * w5x5_ref[cg, u * 5 + v]
        s3c = _leaky(acc5 + b5x5_ref[cg])
        out_ref[c] = sig0 * s1_ref[c] + sig1 * s2c + sig2 * s3c
        return carry

    jax.lax.fori_loop(0, CBLK, cbody, 0)


def _run_a2(s1, t2, t3, w3x3, b3x3, w5x5, b5x5, sig):
    img = lambda: pl.BlockSpec((CBLK, H, W), lambda i: (i, 0, 0))
    smem = lambda shape: pl.BlockSpec(
        shape, lambda i: tuple(0 for _ in shape), memory_space=pltpu.SMEM)
    return pl.pallas_call(
        _a2_body,
        grid=(NCB,),
        in_specs=[
            img(), img(), img(),
            smem((NF, 9)), smem((NF,)), smem((NF, 25)), smem((NF,)),
            smem((3,)),
        ],
        out_specs=img(),
        out_shape=jax.ShapeDtypeStruct((NF, H, W), jnp.float32),
        scratch_shapes=[
            pltpu.VMEM((CBLK, H + 2, W + 2), jnp.float32),
            pltpu.VMEM((CBLK, H + 4, W + 4), jnp.float32),
        ],
        compiler_params=pltpu.CompilerParams(
            vmem_limit_bytes=100 * 1024 * 1024),
    )(s1, t2, t3, w3x3, b3x3, w5x5, b5x5, sig)


def _shift(v, off):
    if off == 0:
        return v
    if off > 0:
        return jnp.concatenate(
            [v[..., off:], jnp.zeros_like(v[..., :off])], axis=-1)
    return jnp.concatenate(
        [jnp.zeros_like(v[..., off:]), v[..., :off]], axis=-1)


def _b_body(pfp_ref, mask_ref, valid_ref, q_ref, spf_ref):
    mask = mask_ref[...]
    cnt = jnp.sum(mask, axis=0, keepdims=True)
    inv0 = 1.0 / jnp.maximum(cnt, 1.0)

    def init_c(c, carry):
        row = jnp.sum(pfp_ref[c], axis=0, keepdims=True) * inv0
        spf_ref[pl.ds(c, 1), :] = row
        return carry

    jax.lax.fori_loop(0, C_TOT, init_c, 0)

    def p2_c(c, acc):
        pc = pfp_ref[c]
        return acc + pc * pc

    p2 = jax.lax.fori_loop(0, C_TOT, p2_c, jnp.zeros((S, K), jnp.float32))

    def one_iter(it, carry):
        def s2_c(c, acc):
            row = spf_ref[pl.ds(c, 1), :]
            return acc + row * row

        s2 = jax.lax.fori_loop(0, C_TOT, s2_c, jnp.zeros((1, K), jnp.float32))

        m = None
        for j in range(9):
            def dot_c(c, acc, _j=j):
                return acc + pfp_ref[c] * _shift(
                    spf_ref[pl.ds(c, 1), :], OFFS[_j])

            dot = jax.lax.fori_loop(0, C_TOT, dot_c,
                                    jnp.zeros((S, K), jnp.float32))
            lj = jnp.where(valid_ref[j:j + 1, :] > 0,
                           2.0 * dot - p2 - _shift(s2, OFFS[j]), -1e16)
            q_ref[j] = lj
            m = lj if m is None else jnp.maximum(m, lj)

        z = jnp.zeros((S, K), jnp.float32)
        for j in range(9):
            e = jnp.exp(q_ref[j] - m)
            q_ref[j] = e
            z = z + e
        invz = 1.0 / z
        for j in range(9):
            q_ref[j] = q_ref[j] * invz * mask

        den = jnp.zeros((1, K), jnp.float32)
        for j in range(9):
            den = den + _shift(jnp.sum(q_ref[j], axis=0, keepdims=True),
                               -OFFS[j])
        invden = 1.0 / (den + 1e-16)

        def num_c(c, carry2):
            pc = pfp_ref[c]
            row = jnp.zeros((1, K), jnp.float32)
            for j in range(9):
                row = row + _shift(
                    jnp.sum(pc * q_ref[j], axis=0, keepdims=True), -OFFS[j])
            spf_ref[pl.ds(c, 1), :] = row * invden
            return carry2

        jax.lax.fori_loop(0, C_TOT, num_c, 0)
        return carry

    jax.lax.fori_loop(0, N_ITERS, one_iter, 0)


def _run_b(pfp, mask, valid):
    return pl.pallas_call(
        _b_body,
        out_shape=[
            jax.ShapeDtypeStruct((9, S, K), jnp.float32),
            jax.ShapeDtypeStruct((C_TOT, K), jnp.float32),
        ],
        compiler_params=pltpu.CompilerParams(
            vmem_limit_bytes=120 * 1024 * 1024),
    )(pfp, mask, valid)


def _to_slot_cell(img):
    ry = jnp.asarray(RY)
    rx = jnp.asarray(RX)
    t = jnp.einsum('rky,cyx->crkx', ry, img,
                   preferred_element_type=jnp.float32,
                   precision=jax.lax.Precision.HIGHEST)
    p = jnp.einsum('sqx,crkx->crskq', rx, t,
                   preferred_element_type=jnp.float32,
                   precision=jax.lax.Precision.HIGHEST)
    return p.reshape(img.shape[0], S, K)


def _from_slot_cell(sc):
    ry = jnp.asarray(RY)
    rx = jnp.asarray(RX)
    p = sc.reshape(sc.shape[0], SLOT, SLOT, NH, NW)
    t = jnp.einsum('jrskq,sqx->jrkx', p, rx,
                   preferred_element_type=jnp.float32,
                   precision=jax.lax.Precision.HIGHEST)
    return jnp.einsum('jrkx,rky->jyx', t, ry,
                      preferred_element_type=jnp.float32,
                   precision=jax.lax.Precision.HIGHEST)


@jax.jit
def kernel(x, params):
    f32 = jnp.float32

    def fold(w, g, b):
        w2 = w[:, :, 0, 0]
        return (w2 * g[None, :]).astype(f32), (w2 @ b)[:, None].astype(f32)

    w0, b0 = fold(params['stem_w'], params['stem_bn_g'], params['stem_bn_b'])
    w1, b1 = fold(params['s1_w'], params['s1_bn_g'], params['s1_bn_b'])
    w2, b2 = fold(params['s2_pw_w'], params['s2_bn_g'], params['s2_bn_b'])
    w3, b3 = fold(params['s3_pw_w'], params['s3_bn_g'], params['s3_bn_b'])

    x2d = x.reshape(IN_CH, PIX).astype(f32)
    s1, t2, t3 = _run_a1(x2d, w0, b0, w1, b1, w2, b2, w3, b3)

    w3x3 = params['s2_dw_w'].reshape(NF, 9).astype(f32)
    w5x5 = params['s3_dw_w'].reshape(NF, 25).astype(f32)
    sig = jnp.concatenate([params['sigma0'], params['sigma1'],
                           params['sigma2']]).astype(f32)
    pf_img = _run_a2(s1.reshape(NF, H, W), t2.reshape(NF, H, W),
                     t3.reshape(NF, H, W), w3x3,
                     params['s2_dw_b'].astype(f32), w5x5,
                     params['s3_dw_b'].astype(f32), sig)

    pf130 = jnp.concatenate(
        [pf_img, jnp.asarray(COORDS[0], f32)], axis=0)
    pfp = _to_slot_cell(pf130)

    qp, spf = _run_b(pfp, jnp.asarray(MASK_SK, f32),
                     jnp.asarray(VALID_9K, f32))

    q_img = _from_slot_cell(qp)[None]
    spf_out = spf[None, :NF, :]
    return q_img, spf_out, pf_img[None]


# speedup vs baseline: 15.6469x; 1.1753x over previous
import functools
import numpy as np
import jax
import jax.numpy as jnp
from jax.experimental import pallas as pl
from jax.experimental.pallas import tpu as pltpu

B, IN_CH, H, W = 1, 202, 256, 256
NF = 128
N_ITERS = 10
N_SPIX = 2621

_nw = int(np.sqrt(N_SPIX * W / H) + 0.5)
_nh = int(np.sqrt(N_SPIX * H / W) + 0.5)
NW, NH = _nw, _nh
K = NW * NH
WS, HS = W / NW, H / NH

_cx_of_x = np.clip((np.arange(W) / WS).astype(np.int32), 0, NW - 1)
_cy_of_y = np.clip((np.arange(H) / HS).astype(np.int32), 0, NH - 1)
_wx = np.bincount(_cx_of_x, minlength=NW)
_wy = np.bincount(_cy_of_y, minlength=NH)
assert _wx[0] == 6 and np.all(_wx[1:] == 5), _wx
assert _wy[0] == 6 and np.all(_wy[1:] == 5), _wy
SLOT = 6
S = SLOT * SLOT

_mrow = np.zeros((NH, SLOT), np.float32)
for _i in range(NH):
    _mrow[_i, : _wy[_i]] = 1.0
_mcol = np.zeros((NW, SLOT), np.float32)
for _i in range(NW):
    _mcol[_i, : _wx[_i]] = 1.0
MASK_SK = np.einsum('yr,xs->rsyx', _mrow, _mcol).reshape(S, K)

_cyk, _cxk = np.arange(K) // NW, np.arange(K) % NW
VALID_9K = np.zeros((9, K), np.float32)
OFFS = []
_j = 0
for _dy in (-1, 0, 1):
    for _dx in (-1, 0, 1):
        VALID_9K[_j] = (((_cyk + _dy) >= 0) & ((_cyk + _dy) < NH)
                        & ((_cxk + _dx) >= 0) & ((_cxk + _dx) < NW))
        OFFS.append(_dy * NW + _dx)
        _j += 1

_ystart = np.zeros(NH, np.int64)
for _i in range(1, NH):
    _ystart[_i] = _ystart[_i - 1] + _wy[_i - 1]
_xstart = np.zeros(NW, np.int64)
for _i in range(1, NW):
    _xstart[_i] = _xstart[_i - 1] + _wx[_i - 1]
RY = np.zeros((SLOT, NH, H), np.float32)
for _cy in range(NH):
    for _r in range(_wy[_cy]):
        RY[_r, _cy, _ystart[_cy] + _r] = 1.0
RX = np.zeros((SLOT, NW, W), np.float32)
for _cx in range(NW):
    for _r in range(_wx[_cx]):
        RX[_r, _cx, _xstart[_cx] + _r] = 1.0

_yv, _xv = np.meshgrid(np.arange(H, dtype=np.float32),
                       np.arange(W, dtype=np.float32), indexing='ij')
COORDS = np.stack([_xv * NW / W, _yv * NH / H])[None]

C_TOT = NF + 2

_LEAK = 0.01


def _leaky(v):
    return jnp.where(v >= 0, v, _LEAK * v)


PIX = H * W
PBLK = 4096
NPB = PIX // PBLK


def _a1_body(x_ref, w0_ref, b0_ref, w1_ref, b1_ref, w2_ref, b2_ref,
             w3_ref, b3_ref, s1_ref, t2_ref, t3_ref):
    xb = x_ref[...]
    s = _leaky(jnp.dot(w0_ref[...], xb, preferred_element_type=jnp.float32)
               + b0_ref[...])
    s1_ref[...] = _leaky(
        jnp.dot(w1_ref[...], s, preferred_element_type=jnp.float32)
        + b1_ref[...])
    t2_ref[...] = (jnp.dot(w2_ref[...], s, preferred_element_type=jnp.float32)
                   + b2_ref[...])
    t3_ref[...] = (jnp.dot(w3_ref[...], s, preferred_element_type=jnp.float32)
                   + b3_ref[...])


def _run_a1(x2d, w0, b0, w1, b1, w2, b2, w3, b3):
    full = lambda shape: pl.BlockSpec(shape, lambda i: (0, 0))
    return pl.pallas_call(
        _a1_body,
        grid=(NPB,),
        in_specs=[
            pl.BlockSpec((IN_CH, PBLK), lambda i: (0, i)),
            full((NF, IN_CH)), full((NF, 1)),
            full((NF, NF)), full((NF, 1)),
            full((NF, NF)), full((NF, 1)),
            full((NF, NF)), full((NF, 1)),
        ],
        out_specs=[pl.BlockSpec((NF, PBLK), lambda i: (0, i))] * 3,
        out_shape=[jax.ShapeDtypeStruct((NF, PIX), jnp.float32)] * 3,
        compiler_params=pltpu.CompilerParams(
            vmem_limit_bytes=100 * 1024 * 1024),
    )(x2d, w0, b0, w1, b1, w2, b2, w3, b3)


CBLK = 16
NCB = NF // CBLK


def _a2_body(s1_ref, t2_ref, t3_ref, w3x3_ref, b3x3_ref, w5x5_ref, b5x5_ref,
             sig_ref, out_ref, scr3, scr5):
    scr3[...] = jnp.zeros_like(scr3)
    scr5[...] = jnp.zeros_like(scr5)
    scr3[:, 1:H + 1, 1:W + 1] = t2_ref[...]
    scr5[:, 2:H + 2, 2:W + 2] = t3_ref[...]
    cbase = pl.program_id(0) * CBLK
    sig0, sig1, sig2 = sig_ref[0], sig_ref[1], sig_ref[2]

    def cbody(c, carry):
        cg = cbase + c
        acc3 = jnp.zeros((H, W), jnp.float32)
        for u in range(3):
            for v in range(3):
                acc3 = acc3 + scr3[c, u:u + H, v:v + W] * w3x3_ref[cg, u * 3 + v]
        s2c = _leaky(acc3 + b3x3_ref[cg])
        acc5 = jnp.zeros((H, W), jnp.float32)
        for u in range(5):
            for v in range(5):
                acc5 = acc5 + scr5[c, u:u + H, v:v + W] * w5x5_ref[cg, u * 5 + v]
        s3c = _leaky(acc5 + b5x5_ref[cg])
        out_ref[c] = sig0 * s1_ref[c] + sig1 * s2c + sig2 * s3c
        return carry

    jax.lax.fori_loop(0, CBLK, cbody, 0)


def _run_a2(s1, t2, t3, w3x3, b3x3, w5x5, b5x5, sig):
    img = lambda: pl.BlockSpec((CBLK, H, W), lambda i: (i, 0, 0))
    smem = lambda shape: pl.BlockSpec(
        shape, lambda i: tuple(0 for _ in shape), memory_space=pltpu.SMEM)
    return pl.pallas_call(
        _a2_body,
        grid=(NCB,),
        in_specs=[
            img(), img(), img(),
            smem((NF, 9)), smem((NF,)), smem((NF, 25)), smem((NF,)),
            smem((3,)),
        ],
        out_specs=img(),
        out_shape=jax.ShapeDtypeStruct((NF, H, W), jnp.float32),
        scratch_shapes=[
            pltpu.VMEM((CBLK, H + 2, W + 2), jnp.float32),
            pltpu.VMEM((CBLK, H + 4, W + 4), jnp.float32),
        ],
        compiler_params=pltpu.CompilerParams(
            vmem_limit_bytes=100 * 1024 * 1024),
    )(s1, t2, t3, w3x3, b3x3, w5x5, b5x5, sig)


def _shift(v, off):
    if off == 0:
        return v
    if off > 0:
        return jnp.concatenate(
            [v[..., off:], jnp.zeros_like(v[..., :off])], axis=-1)
    return jnp.concatenate(
        [jnp.zeros_like(v[..., off:]), v[..., :off]], axis=-1)


def _b_body(pfp_ref, mask_ref, valid_ref, q_ref, spf_ref):
    mask = mask_ref[...]
    cnt = jnp.sum(mask, axis=0, keepdims=True)
    inv0 = 1.0 / jnp.maximum(cnt, 1.0)

    def init_c(c, carry):
        row = jnp.sum(pfp_ref[c], axis=0, keepdims=True) * inv0
        spf_ref[pl.ds(c, 1), :] = row
        return carry

    jax.lax.fori_loop(0, C_TOT, init_c, 0)

    def one_iter(it, carry):
        def s2_c(c, acc):
            row = spf_ref[pl.ds(c, 1), :]
            return acc + row * row

        s2 = jax.lax.fori_loop(0, C_TOT, s2_c, jnp.zeros((1, K), jnp.float32))

        m = None
        for grp in ((0, 1, 2), (3, 4, 5), (6, 7, 8)):
            def dot_c(c, accs, _grp=grp):
                pc = pfp_ref[c]
                srow = spf_ref[pl.ds(c, 1), :]
                return tuple(accs[i] + pc * _shift(srow, OFFS[_grp[i]])
                             for i in range(3))

            dots = jax.lax.fori_loop(
                0, C_TOT, dot_c,
                tuple(jnp.zeros((S, K), jnp.float32) for _ in range(3)))
            for i, j in enumerate(grp):
                lj = jnp.where(valid_ref[j:j + 1, :] > 0,
                               2.0 * dots[i] - _shift(s2, OFFS[j]), -1e16)
                q_ref[j] = lj
                m = lj if m is None else jnp.maximum(m, lj)

        z = jnp.zeros((S, K), jnp.float32)
        for j in range(9):
            e = jnp.exp(q_ref[j] - m)
            q_ref[j] = e
            z = z + e
        invz = 1.0 / z
        for j in range(9):
            q_ref[j] = q_ref[j] * invz * mask

        den = jnp.zeros((1, K), jnp.float32)
        for j in range(9):
            den = den + _shift(jnp.sum(q_ref[j], axis=0, keepdims=True),
                               -OFFS[j])
        invden = 1.0 / (den + 1e-16)

        def num_c(c, carry2):
            pc = pfp_ref[c]
            row = jnp.zeros((1, K), jnp.float32)
            for j in range(9):
                row = row + _shift(
                    jnp.sum(pc * q_ref[j], axis=0, keepdims=True), -OFFS[j])
            spf_ref[pl.ds(c, 1), :] = row * invden
            return carry2

        jax.lax.fori_loop(0, C_TOT, num_c, 0)
        return carry

    jax.lax.fori_loop(0, N_ITERS, one_iter, 0)


def _run_b(pfp, mask, valid):
    return pl.pallas_call(
        _b_body,
        out_shape=[
            jax.ShapeDtypeStruct((9, S, K), jnp.float32),
            jax.ShapeDtypeStruct((C_TOT, K), jnp.float32),
        ],
        compiler_params=pltpu.CompilerParams(
            vmem_limit_bytes=120 * 1024 * 1024),
    )(pfp, mask, valid)


def _to_slot_cell(img):
    ry = jnp.asarray(RY)
    rx = jnp.asarray(RX)
    t = jnp.einsum('rky,cyx->crkx', ry, img,
                   preferred_element_type=jnp.float32,
                   precision=jax.lax.Precision.HIGHEST)
    p = jnp.einsum('sqx,crkx->crskq', rx, t,
                   preferred_element_type=jnp.float32,
                   precision=jax.lax.Precision.HIGHEST)
    return p.reshape(img.shape[0], S, K)


def _from_slot_cell(sc):
    ry = jnp.asarray(RY)
    rx = jnp.asarray(RX)
    p = sc.reshape(sc.shape[0], SLOT, SLOT, NH, NW)
    t = jnp.einsum('jrskq,sqx->jrkx', p, rx,
                   preferred_element_type=jnp.float32,
                   precision=jax.lax.Precision.HIGHEST)
    return jnp.einsum('jrkx,rky->jyx', t, ry,
                      preferred_element_type=jnp.float32,
                   precision=jax.lax.Precision.HIGHEST)


@jax.jit
def kernel(x, params):
    f32 = jnp.float32

    def fold(w, g, b):
        w2 = w[:, :, 0, 0]
        return (w2 * g[None, :]).astype(f32), (w2 @ b)[:, None].astype(f32)

    w0, b0 = fold(params['stem_w'], params['stem_bn_g'], params['stem_bn_b'])
    w1, b1 = fold(params['s1_w'], params['s1_bn_g'], params['s1_bn_b'])
    w2, b2 = fold(params['s2_pw_w'], params['s2_bn_g'], params['s2_bn_b'])
    w3, b3 = fold(params['s3_pw_w'], params['s3_bn_g'], params['s3_bn_b'])

    x2d = x.reshape(IN_CH, PIX).astype(f32)
    s1, t2, t3 = _run_a1(x2d, w0, b0, w1, b1, w2, b2, w3, b3)

    w3x3 = params['s2_dw_w'].reshape(NF, 9).astype(f32)
    w5x5 = params['s3_dw_w'].reshape(NF, 25).astype(f32)
    sig = jnp.concatenate([params['sigma0'], params['sigma1'],
                           params['sigma2']]).astype(f32)
    pf_img = _run_a2(s1.reshape(NF, H, W), t2.reshape(NF, H, W),
                     t3.reshape(NF, H, W), w3x3,
                     params['s2_dw_b'].astype(f32), w5x5,
                     params['s3_dw_b'].astype(f32), sig)

    pf130 = jnp.concatenate(
        [pf_img, jnp.asarray(COORDS[0], f32)], axis=0)
    pfp = _to_slot_cell(pf130)

    qp, spf = _run_b(pfp, jnp.asarray(MASK_SK, f32),
                     jnp.asarray(VALID_9K, f32))

    q_img = _from_slot_cell(qp)[None]
    spf_out = spf[None, :NF, :]
    return q_img, spf_out, pf_img[None]


# speedup vs baseline: 16.0916x; 1.0284x over previous
import functools
import numpy as np
import jax
import jax.numpy as jnp
from jax.experimental import pallas as pl
from jax.experimental.pallas import tpu as pltpu

B, IN_CH, H, W = 1, 202, 256, 256
NF = 128
N_ITERS = 10
N_SPIX = 2621

_nw = int(np.sqrt(N_SPIX * W / H) + 0.5)
_nh = int(np.sqrt(N_SPIX * H / W) + 0.5)
NW, NH = _nw, _nh
K = NW * NH
WS, HS = W / NW, H / NH

_cx_of_x = np.clip((np.arange(W) / WS).astype(np.int32), 0, NW - 1)
_cy_of_y = np.clip((np.arange(H) / HS).astype(np.int32), 0, NH - 1)
_wx = np.bincount(_cx_of_x, minlength=NW)
_wy = np.bincount(_cy_of_y, minlength=NH)
assert _wx[0] == 6 and np.all(_wx[1:] == 5), _wx
assert _wy[0] == 6 and np.all(_wy[1:] == 5), _wy
SLOT = 6
S = SLOT * SLOT

_mrow = np.zeros((NH, SLOT), np.float32)
for _i in range(NH):
    _mrow[_i, : _wy[_i]] = 1.0
_mcol = np.zeros((NW, SLOT), np.float32)
for _i in range(NW):
    _mcol[_i, : _wx[_i]] = 1.0
MASK_SK = np.einsum('yr,xs->rsyx', _mrow, _mcol).reshape(S, K)

_cyk, _cxk = np.arange(K) // NW, np.arange(K) % NW
VALID_9K = np.zeros((9, K), np.float32)
OFFS = []
_j = 0
for _dy in (-1, 0, 1):
    for _dx in (-1, 0, 1):
        VALID_9K[_j] = (((_cyk + _dy) >= 0) & ((_cyk + _dy) < NH)
                        & ((_cxk + _dx) >= 0) & ((_cxk + _dx) < NW))
        OFFS.append(_dy * NW + _dx)
        _j += 1

_ystart = np.zeros(NH, np.int64)
for _i in range(1, NH):
    _ystart[_i] = _ystart[_i - 1] + _wy[_i - 1]
_xstart = np.zeros(NW, np.int64)
for _i in range(1, NW):
    _xstart[_i] = _xstart[_i - 1] + _wx[_i - 1]
RY = np.zeros((SLOT, NH, H), np.float32)
for _cy in range(NH):
    for _r in range(_wy[_cy]):
        RY[_r, _cy, _ystart[_cy] + _r] = 1.0
RX = np.zeros((SLOT, NW, W), np.float32)
for _cx in range(NW):
    for _r in range(_wx[_cx]):
        RX[_r, _cx, _xstart[_cx] + _r] = 1.0

_yv, _xv = np.meshgrid(np.arange(H, dtype=np.float32),
                       np.arange(W, dtype=np.float32), indexing='ij')
COORDS = np.stack([_xv * NW / W, _yv * NH / H])[None]

C_TOT = NF + 2

_LEAK = 0.01


def _leaky(v):
    return jnp.where(v >= 0, v, _LEAK * v)


PIX = H * W
PBLK = 4096
NPB = PIX // PBLK


def _a1_body(x_ref, w0_ref, b0_ref, w1_ref, b1_ref, w2_ref, b2_ref,
             w3_ref, b3_ref, s1_ref, t2_ref, t3_ref):
    xb = x_ref[...]
    s = _leaky(jnp.dot(w0_ref[...], xb, preferred_element_type=jnp.float32)
               + b0_ref[...])
    s1_ref[...] = _leaky(
        jnp.dot(w1_ref[...], s, preferred_element_type=jnp.float32)
        + b1_ref[...])
    t2_ref[...] = (jnp.dot(w2_ref[...], s, preferred_element_type=jnp.float32)
                   + b2_ref[...])
    t3_ref[...] = (jnp.dot(w3_ref[...], s, preferred_element_type=jnp.float32)
                   + b3_ref[...])


def _run_a1(x2d, w0, b0, w1, b1, w2, b2, w3, b3):
    full = lambda shape: pl.BlockSpec(shape, lambda i: (0, 0))
    return pl.pallas_call(
        _a1_body,
        grid=(NPB,),
        in_specs=[
            pl.BlockSpec((IN_CH, PBLK), lambda i: (0, i)),
            full((NF, IN_CH)), full((NF, 1)),
            full((NF, NF)), full((NF, 1)),
            full((NF, NF)), full((NF, 1)),
            full((NF, NF)), full((NF, 1)),
        ],
        out_specs=[pl.BlockSpec((NF, PBLK), lambda i: (0, i))] * 3,
        out_shape=[jax.ShapeDtypeStruct((NF, PIX), jnp.float32)] * 3,
        compiler_params=pltpu.CompilerParams(
            vmem_limit_bytes=100 * 1024 * 1024),
    )(x2d, w0, b0, w1, b1, w2, b2, w3, b3)


CBLK = 16
NCB = NF // CBLK


def _a2_body(s1_ref, t2_ref, t3_ref, w3x3_ref, b3x3_ref, w5x5_ref, b5x5_ref,
             sig_ref, out_ref, scr3, scr5):
    scr3[...] = jnp.zeros_like(scr3)
    scr5[...] = jnp.zeros_like(scr5)
    scr3[:, 1:H + 1, 1:W + 1] = t2_ref[...]
    scr5[:, 2:H + 2, 2:W + 2] = t3_ref[...]
    cbase = pl.program_id(0) * CBLK
    sig0, sig1, sig2 = sig_ref[0], sig_ref[1], sig_ref[2]

    def cbody(c, carry):
        cg = cbase + c
        acc3 = jnp.zeros((H, W), jnp.float32)
        for u in range(3):
            for v in range(3):
                acc3 = acc3 + scr3[c, u:u + H, v:v + W] * w3x3_ref[cg, u * 3 + v]
        s2c = _leaky(acc3 + b3x3_ref[cg])
        acc5 = jnp.zeros((H, W), jnp.float32)
        for u in range(5):
            for v in range(5):
                acc5 = acc5 + scr5[c, u:u + H, v:v + W] * w5x5_ref[cg, u * 5 + v]
        s3c = _leaky(acc5 + b5x5_ref[cg])
        out_ref[c] = sig0 * s1_ref[c] + sig1 * s2c + sig2 * s3c
        return carry

    jax.lax.fori_loop(0, CBLK, cbody, 0)


def _run_a2(s1, t2, t3, w3x3, b3x3, w5x5, b5x5, sig):
    img = lambda: pl.BlockSpec((CBLK, H, W), lambda i: (i, 0, 0))
    smem = lambda shape: pl.BlockSpec(
        shape, lambda i: tuple(0 for _ in shape), memory_space=pltpu.SMEM)
    return pl.pallas_call(
        _a2_body,
        grid=(NCB,),
        in_specs=[
            img(), img(), img(),
            smem((NF, 9)), smem((NF,)), smem((NF, 25)), smem((NF,)),
            smem((3,)),
        ],
        out_specs=img(),
        out_shape=jax.ShapeDtypeStruct((NF, H, W), jnp.float32),
        scratch_shapes=[
            pltpu.VMEM((CBLK, H + 2, W + 2), jnp.float32),
            pltpu.VMEM((CBLK, H + 4, W + 4), jnp.float32),
        ],
        compiler_params=pltpu.CompilerParams(
            vmem_limit_bytes=100 * 1024 * 1024),
    )(s1, t2, t3, w3x3, b3x3, w5x5, b5x5, sig)


def _shift(v, off):
    if off == 0:
        return v
    if off > 0:
        return jnp.concatenate(
            [v[..., off:], jnp.zeros_like(v[..., :off])], axis=-1)
    return jnp.concatenate(
        [jnp.zeros_like(v[..., off:]), v[..., :off]], axis=-1)


def _b_body(pfp_ref, mask_ref, valid_ref, q_ref, spf_ref):
    mask = mask_ref[...]
    cnt = jnp.sum(mask, axis=0, keepdims=True)
    inv0 = 1.0 / jnp.maximum(cnt, 1.0)

    def init_c(c, carry):
        row = jnp.sum(pfp_ref[c], axis=0, keepdims=True) * inv0
        spf_ref[pl.ds(c, 1), :] = row
        return carry

    jax.lax.fori_loop(0, C_TOT, init_c, 0)

    def one_iter(it, carry):
        def s2_c(c, acc):
            row = spf_ref[pl.ds(c, 1), :]
            return acc + row * row

        s2 = jax.lax.fori_loop(0, C_TOT, s2_c, jnp.zeros((1, K), jnp.float32))

        m = None
        for grp in ((0, 1, 2), (3, 4, 5), (6, 7, 8)):
            def dot_c(c, accs, _grp=grp):
                pc = pfp_ref[c]
                srow = spf_ref[pl.ds(c, 1), :]
                return tuple(accs[i] + pc * _shift(srow, OFFS[_grp[i]])
                             for i in range(3))

            dots = jax.lax.fori_loop(
                0, C_TOT, dot_c,
                tuple(jnp.zeros((S, K), jnp.float32) for _ in range(3)))
            for i, j in enumerate(grp):
                lj = jnp.where(valid_ref[j:j + 1, :] > 0,
                               2.0 * dots[i] - _shift(s2, OFFS[j]), -1e16)
                q_ref[j] = lj
                m = lj if m is None else jnp.maximum(m, lj)

        z = jnp.zeros((S, K), jnp.float32)
        for j in range(9):
            e = jnp.exp(q_ref[j] - m)
            q_ref[j] = e
            z = z + e
        invz = 1.0 / z
        for j in range(9):
            q_ref[j] = q_ref[j] * invz * mask

        den = jnp.zeros((1, K), jnp.float32)
        for j in range(9):
            den = den + _shift(jnp.sum(q_ref[j], axis=0, keepdims=True),
                               -OFFS[j])
        invden = 1.0 / (den + 1e-16)

        def num_c(c, carry2):
            pc = pfp_ref[c]
            row = jnp.zeros((1, K), jnp.float32)
            for j in range(9):
                row = row + _shift(
                    jnp.sum(pc * q_ref[j], axis=0, keepdims=True), -OFFS[j])
            spf_ref[pl.ds(c, 1), :] = row * invden
            return carry2

        jax.lax.fori_loop(0, C_TOT, num_c, 0)
        return carry

    jax.lax.fori_loop(0, N_ITERS, one_iter, 0)


def _run_b(pfp, mask, valid):
    return pl.pallas_call(
        _b_body,
        out_shape=[
            jax.ShapeDtypeStruct((9, S, K), jnp.float32),
            jax.ShapeDtypeStruct((C_TOT, K), jnp.float32),
        ],
        compiler_params=pltpu.CompilerParams(
            vmem_limit_bytes=120 * 1024 * 1024),
    )(pfp, mask, valid)


def _to_slot_cell(img):
    ry = jnp.asarray(RY)
    rx = jnp.asarray(RX)
    t = jnp.einsum('rky,cyx->crkx', ry, img,
                   preferred_element_type=jnp.float32,
                   precision=jax.lax.Precision.HIGH)
    p = jnp.einsum('sqx,crkx->crskq', rx, t,
                   preferred_element_type=jnp.float32,
                   precision=jax.lax.Precision.HIGH)
    return p.reshape(img.shape[0], S, K)


def _from_slot_cell(sc):
    ry = jnp.asarray(RY)
    rx = jnp.asarray(RX)
    p = sc.reshape(sc.shape[0], SLOT, SLOT, NH, NW)
    t = jnp.einsum('jrskq,sqx->jrkx', p, rx,
                   preferred_element_type=jnp.float32,
                   precision=jax.lax.Precision.HIGH)
    return jnp.einsum('jrkx,rky->jyx', t, ry,
                      preferred_element_type=jnp.float32,
                   precision=jax.lax.Precision.HIGH)


@jax.jit
def kernel(x, params):
    f32 = jnp.float32

    def fold(w, g, b):
        w2 = w[:, :, 0, 0]
        return (w2 * g[None, :]).astype(f32), (w2 @ b)[:, None].astype(f32)

    w0, b0 = fold(params['stem_w'], params['stem_bn_g'], params['stem_bn_b'])
    w1, b1 = fold(params['s1_w'], params['s1_bn_g'], params['s1_bn_b'])
    w2, b2 = fold(params['s2_pw_w'], params['s2_bn_g'], params['s2_bn_b'])
    w3, b3 = fold(params['s3_pw_w'], params['s3_bn_g'], params['s3_bn_b'])

    x2d = x.reshape(IN_CH, PIX).astype(f32)
    s1, t2, t3 = _run_a1(x2d, w0, b0, w1, b1, w2, b2, w3, b3)

    w3x3 = params['s2_dw_w'].reshape(NF, 9).astype(f32)
    w5x5 = params['s3_dw_w'].reshape(NF, 25).astype(f32)
    sig = jnp.concatenate([params['sigma0'], params['sigma1'],
                           params['sigma2']]).astype(f32)
    pf_img = _run_a2(s1.reshape(NF, H, W), t2.reshape(NF, H, W),
                     t3.reshape(NF, H, W), w3x3,
                     params['s2_dw_b'].astype(f32), w5x5,
                     params['s3_dw_b'].astype(f32), sig)

    pf130 = jnp.concatenate(
        [pf_img, jnp.asarray(COORDS[0], f32)], axis=0)
    pfp = _to_slot_cell(pf130)

    qp, spf = _run_b(pfp, jnp.asarray(MASK_SK, f32),
                     jnp.asarray(VALID_9K, f32))

    q_img = _from_slot_cell(qp)[None]
    spf_out = spf[None, :NF, :]
    return q_img, spf_out, pf_img[None]
